# baseline (speedup 1.0000x reference)
import functools

import jax
import jax.numpy as jnp
from jax import lax
from jax.experimental import pallas as pl
from jax.experimental.pallas import tpu as pltpu
from jax.experimental.pallas import tpu_sc as plsc

N = 10000
E = 320000
D = 128

NC = 2
NS = 16
NW = NC * NS
EPW = E // NW
K = 80
NCHUNK = EPW // K
NP = 10240
RPT = NP // NS

NBUF = 3

_mesh = plsc.VectorSubcoreMesh(
    core_axis_name="c", subcore_axis_name="s", num_cores=NC, num_subcores=NS
)


def _unpack_idx(pk_v, c, src_buf, dst_buf):
    for j in range(K // 16):
        v = pk_v[c, pl.ds(16 * j, 16)]
        src_buf[pl.ds(16 * j, 16)] = lax.shift_right_logical(v, 16)
        dst_buf[pl.ds(16 * j, 16)] = lax.bitwise_and(v, 0xFFFF)


@functools.partial(
    pl.kernel,
    out_type=jax.ShapeDtypeStruct((NC, NP, D), jnp.float32),
    mesh=_mesh,
    scratch_types=[
        pltpu.VMEM((NCHUNK, K), jnp.int32),
        [pltpu.VMEM((K,), jnp.int32)] * NBUF,
        [pltpu.VMEM((K,), jnp.int32)] * NBUF,
        [pltpu.VMEM((K, D), jnp.float32)] * NBUF,
        pltpu.VMEM_SHARED((NP, D), jnp.float32),
        [pltpu.SemaphoreType.DMA] * NBUF,
        [pltpu.SemaphoreType.DMA] * NBUF,
        pltpu.SemaphoreType.DMA,
        pltpu.SemaphoreType.DMA,
    ],
)
def _sc_segment_sum(y_hbm, pk_hbm, out_hbm,
                    pk_v, srcb, dstb, rows, acc_sh, gsem, ssem, psem, zsem):
    cid = lax.axis_index("c")
    sid = lax.axis_index("s")
    wid = sid * NC + cid

    pltpu.async_copy(pk_hbm.at[wid], pk_v, psem)

    zer = jnp.zeros((16,), jnp.float32)

    def zrow(r, carry):
        for j in range(D // 16):
            rows[0][r, pl.ds(16 * j, 16)] = zer
        return carry

    lax.fori_loop(0, K, zrow, 0)
    for t in range(RPT // K):
        pltpu.async_copy(rows[0], acc_sh.at[pl.ds(sid * RPT + t * K, K)], zsem)

    pltpu.make_async_copy(pk_hbm.at[wid], pk_v, psem).wait()
    for t in range(RPT // K):
        pltpu.make_async_copy(
            rows[0], acc_sh.at[pl.ds(sid * RPT + t * K, K)], zsem).wait()

    for b in range(NBUF):
        _unpack_idx(pk_v, b, srcb[b], dstb[b])
        pltpu.async_copy(y_hbm.at[srcb[b]], rows[b], gsem[b])
    plsc.subcore_barrier()

    def chunk_step(c, b, prefetch):
        pltpu.make_async_copy(y_hbm.at[srcb[b]], rows[b], gsem[b]).wait()
        pltpu.async_copy(rows[b], acc_sh.at[dstb[b]], ssem[b], add=True)
        if prefetch:
            p = (b + NBUF - 1) % NBUF

            @pl.when(jnp.logical_and(c >= 1, c + NBUF - 1 < NCHUNK))
            def _():
                pltpu.make_async_copy(
                    rows[p], acc_sh.at[dstb[p]], ssem[p]).wait()
                _unpack_idx(pk_v, c + NBUF - 1, srcb[p], dstb[p])
                pltpu.async_copy(y_hbm.at[srcb[p]], rows[p], gsem[p])

    def outer(i, carry):
        for b in range(NBUF):
            chunk_step(i * NBUF + b, b, True)
        return carry

    lax.fori_loop(0, NCHUNK // NBUF, outer, 0)

    for c in range(NCHUNK - NCHUNK % NBUF, NCHUNK):
        chunk_step(c, c % NBUF, False)

    for b in range(NBUF):
        pltpu.make_async_copy(rows[b], acc_sh.at[dstb[b]], ssem[b]).wait()

    plsc.subcore_barrier()
    pltpu.sync_copy(acc_sh.at[pl.ds(sid * RPT, RPT)],
                    out_hbm.at[cid].at[pl.ds(sid * RPT, RPT)])


def _tc_body(relu, p_ref, w_ref, b_ref, o_ref):
    s = p_ref[0] + p_ref[1]
    y = jnp.dot(s, w_ref[...], preferred_element_type=jnp.float32) + b_ref[...]
    o_ref[...] = jnp.maximum(y, 0.0) if relu else y


def _tc_layer(p, w, b, relu):
    bn = 10000
    return pl.pallas_call(
        functools.partial(_tc_body, relu),
        grid=(N // bn,),
        in_specs=[
            pl.BlockSpec((2, bn, D), lambda i: (0, i, 0)),
            pl.BlockSpec((D, D), lambda i: (0, 0)),
            pl.BlockSpec((1, D), lambda i: (0, 0)),
        ],
        out_specs=pl.BlockSpec((bn, D), lambda i: (i, 0)),
        out_shape=jax.ShapeDtypeStruct((N, D), jnp.float32),
    )(p, w, b.reshape(1, D))


def kernel(x, edge_index, W1, b1, W2, b2):
    packed = jnp.bitwise_or(
        jnp.left_shift(edge_index[0], 16), edge_index[1]
    ).reshape(NW, NCHUNK, K)
    p1 = _sc_segment_sum(x, packed)
    h1 = _tc_layer(p1, W1, b1, relu=True)
    p2 = _sc_segment_sum(h1, packed)
    return _tc_layer(p2, W2, b2, relu=False)


# speedup vs baseline: 1.0069x; 1.0069x over previous
import functools

import jax
import jax.numpy as jnp
from jax import lax
from jax.experimental import pallas as pl
from jax.experimental.pallas import tpu as pltpu
from jax.experimental.pallas import tpu_sc as plsc

N = 10000
E = 320000
D = 128

NC = 2
NS = 16
NW = NC * NS
EPW = E // NW
K = 80
NCHUNK = EPW // K
NP = 10240
RPT = NP // NS

NBUF = 3

_mesh = plsc.VectorSubcoreMesh(
    core_axis_name="c", subcore_axis_name="s", num_cores=NC, num_subcores=NS
)


def _unpack_idx(pk_v, c, src_buf, dst_buf):
    for j in range(K // 16):
        v = pk_v[c, pl.ds(16 * j, 16)]
        src_buf[pl.ds(16 * j, 16)] = lax.shift_right_logical(v, 16)
        dst_buf[pl.ds(16 * j, 16)] = lax.bitwise_and(v, 0xFFFF)


@functools.partial(
    pl.kernel,
    out_type=jax.ShapeDtypeStruct((NC, NP, D), jnp.float32),
    mesh=_mesh,
    scratch_types=[
        pltpu.VMEM((NCHUNK, K), jnp.int32),
        [pltpu.VMEM((K,), jnp.int32)] * NBUF,
        [pltpu.VMEM((K,), jnp.int32)] * NBUF,
        [pltpu.VMEM((K, D), jnp.float32)] * NBUF,
        pltpu.VMEM_SHARED((NP, D), jnp.float32),
        [pltpu.SemaphoreType.DMA] * NBUF,
        [pltpu.SemaphoreType.DMA] * NBUF,
        pltpu.SemaphoreType.DMA,
        pltpu.SemaphoreType.DMA,
    ],
)
def _sc_segment_sum(y_hbm, pk_hbm, out_hbm,
                    pk_v, srcb, dstb, rows, acc_sh, gsem, ssem, psem, zsem):
    cid = lax.axis_index("c")
    sid = lax.axis_index("s")
    wid = sid * NC + cid

    pltpu.async_copy(pk_hbm.at[wid], pk_v, psem)

    zer = jnp.zeros((16,), jnp.float32)

    def zrow(r, carry):
        for j in range(D // 16):
            rows[0][r, pl.ds(16 * j, 16)] = zer
        return carry

    lax.fori_loop(0, K, zrow, 0)
    for t in range(RPT // K):
        pltpu.async_copy(rows[0], acc_sh.at[pl.ds(sid * RPT + t * K, K)], zsem)

    pltpu.make_async_copy(pk_hbm.at[wid], pk_v, psem).wait()
    for t in range(RPT // K):
        pltpu.make_async_copy(
            rows[0], acc_sh.at[pl.ds(sid * RPT + t * K, K)], zsem).wait()

    for b in range(NBUF):
        _unpack_idx(pk_v, b, srcb[b], dstb[b])
        pltpu.async_copy(y_hbm.at[srcb[b]], rows[b], gsem[b])
    plsc.subcore_barrier()

    def chunk_step(c, b, prefetch):
        pltpu.make_async_copy(y_hbm.at[srcb[b]], rows[b], gsem[b]).wait()
        pltpu.async_copy(rows[b], acc_sh.at[dstb[b]], ssem[b], add=True)
        if prefetch:
            p = (b + NBUF - 1) % NBUF

            @pl.when(jnp.logical_and(c >= 1, c + NBUF - 1 < NCHUNK))
            def _():
                pltpu.make_async_copy(
                    rows[p], acc_sh.at[dstb[p]], ssem[p]).wait()
                _unpack_idx(pk_v, c + NBUF - 1, srcb[p], dstb[p])
                pltpu.async_copy(y_hbm.at[srcb[p]], rows[p], gsem[p])

    def outer(i, carry):
        for b in range(NBUF):
            chunk_step(i * NBUF + b, b, True)
        return carry

    lax.fori_loop(0, NCHUNK // NBUF, outer, 0)

    for c in range(NCHUNK - NCHUNK % NBUF, NCHUNK):
        chunk_step(c, c % NBUF, False)

    for b in range(NBUF):
        pltpu.make_async_copy(rows[b], acc_sh.at[dstb[b]], ssem[b]).wait()

    plsc.subcore_barrier()
    pltpu.sync_copy(acc_sh.at[pl.ds(sid * RPT, RPT)],
                    out_hbm.at[cid].at[pl.ds(sid * RPT, RPT)])


def _tc_body(relu, p_ref, w_ref, b_ref, o_ref):
    s = p_ref[0] + p_ref[1]
    y = jnp.dot(s, w_ref[...], preferred_element_type=jnp.float32) + b_ref[...]
    o_ref[...] = jnp.maximum(y, 0.0) if relu else y


def _tc_layer(p, w, b, relu):
    bn = 5000
    return pl.pallas_call(
        functools.partial(_tc_body, relu),
        grid=(N // bn,),
        in_specs=[
            pl.BlockSpec((2, bn, D), lambda i: (0, i, 0)),
            pl.BlockSpec((D, D), lambda i: (0, 0)),
            pl.BlockSpec((1, D), lambda i: (0, 0)),
        ],
        out_specs=pl.BlockSpec((bn, D), lambda i: (i, 0)),
        out_shape=jax.ShapeDtypeStruct((N, D), jnp.float32),
    )(p, w, b.reshape(1, D))


def kernel(x, edge_index, W1, b1, W2, b2):
    packed = jnp.bitwise_or(
        jnp.left_shift(edge_index[0], 16), edge_index[1]
    ).reshape(NW, NCHUNK, K)
    p1 = _sc_segment_sum(x, packed)
    h1 = _tc_layer(p1, W1, b1, relu=True)
    p2 = _sc_segment_sum(h1, packed)
    return _tc_layer(p2, W2, b2, relu=False)
